# baseline (speedup 1.0000x reference)
import jax
import jax.numpy as jnp
from jax.experimental import pallas as pl
from jax.experimental.pallas import tpu as pltpu

_LOG2E = 1.4426950408889634


def _gat_kernel(x_ref, w_ref, asrc_ref, adst_ref, ei_ref, b_ref, out_ref,
                hbf_ref, es2_ref, es022_ref, edcol_ref, eirow_ref, eicol_ref):
    i = pl.program_id(0)
    blk = out_ref.shape[0]
    C = out_ref.shape[1]

    @pl.when(i == 0)
    def _init():
        h = jnp.dot(x_ref[...], w_ref[...], preferred_element_type=jnp.float32)
        hbf_ref[:, :C] = h.astype(jnp.bfloat16)
        hbf_ref[:, C:] = jnp.ones((h.shape[0], 128), jnp.bfloat16)
        es = jax.lax.dot_general(asrc_ref[...], h, (((1,), (1,)), ((), ())),
                                 preferred_element_type=jnp.float32)
        es2_ref[...] = _LOG2E * es
        es022_ref[...] = (0.2 * _LOG2E) * es
        edcol_ref[...] = jnp.dot(h, adst_ref[...],
                                 preferred_element_type=jnp.float32)
        ei = ei_ref[...]
        eirow_ref[...] = ei.astype(jnp.int16)
        eicol_ref[...] = jnp.transpose(ei.astype(jnp.float32)).astype(jnp.int16)

    ed = edcol_ref[pl.ds(i * blk, blk), :]
    es2 = es2_ref[...]
    smax2 = jnp.max(es2)
    q2 = _LOG2E * ed + smax2
    mi2 = jnp.maximum(q2, 0.2 * q2)
    edm2 = _LOG2E * ed - mi2
    c22 = (0.2 * _LOG2E) * ed - mi2
    si = eicol_ref[pl.ds(i * blk, blk), 0:1]
    di = eicol_ref[pl.ds(i * blk, blk), 1:2]
    sj = eirow_ref[0:1, :]
    dj = eirow_ref[1:2, :]
    zs = jnp.maximum(edm2 + es2, c22 + es022_ref[...])
    conn = ((si == sj) | (si == dj)) | ((di == sj) | (di == dj))
    p = jnp.where(conn, jnp.exp2(zs).astype(jnp.bfloat16), jnp.bfloat16(0.0))
    acc = jnp.dot(p, hbf_ref[...],
                  preferred_element_type=jnp.float32)
    s = acc[:, C:C + 1]
    out_ref[...] = acc[:, :C] * (1.0 / s) + b_ref[...]


def kernel(x, edge_index, W, a_src, a_dst, b):
    E, _ = x.shape
    C = W.shape[1]
    blk = 256
    out = pl.pallas_call(
        _gat_kernel,
        grid=(E // blk,),
        in_specs=[
            pl.BlockSpec((E, x.shape[1]), lambda i: (0, 0)),
            pl.BlockSpec((x.shape[1], C), lambda i: (0, 0)),
            pl.BlockSpec((1, C), lambda i: (0, 0)),
            pl.BlockSpec((C, 1), lambda i: (0, 0)),
            pl.BlockSpec((2, E), lambda i: (0, 0)),
            pl.BlockSpec((1, C), lambda i: (0, 0)),
        ],
        out_specs=pl.BlockSpec((blk, C), lambda i: (i, 0)),
        out_shape=jax.ShapeDtypeStruct((E, C), jnp.float32),
        scratch_shapes=[
            pltpu.VMEM((E, C + 128), jnp.bfloat16),
            pltpu.VMEM((1, E), jnp.float32),
            pltpu.VMEM((1, E), jnp.float32),
            pltpu.VMEM((E, 1), jnp.float32),
            pltpu.VMEM((2, E), jnp.int16),
            pltpu.VMEM((E, 2), jnp.int16),
        ],
    )(x, W, a_src.reshape(1, C), a_dst.reshape(C, 1), edge_index,
      b.reshape(1, C))
    return out


# speedup vs baseline: 1.2111x; 1.2111x over previous
import jax
import jax.numpy as jnp
from jax.experimental import pallas as pl
from jax.experimental.pallas import tpu as pltpu

_LOG2E = 1.4426950408889634


def _gat_kernel(x_ref, w_ref, asrc_ref, adst_ref, ei_ref, b_ref, out_ref,
                hbf_ref, es2_ref, es022_ref, edcol_ref, eirow_ref, eicol_ref):
    i = pl.program_id(0)
    blk = out_ref.shape[0]
    C = out_ref.shape[1]

    @pl.when(i == 0)
    def _init():
        h = jnp.dot(x_ref[...], w_ref[...], preferred_element_type=jnp.float32)
        hbf_ref[:, :C] = h.astype(jnp.bfloat16)
        hbf_ref[:, C:] = jnp.ones((h.shape[0], 128), jnp.bfloat16)
        es = jax.lax.dot_general(asrc_ref[...], h, (((1,), (1,)), ((), ())),
                                 preferred_element_type=jnp.float32)
        es2_ref[...] = _LOG2E * es
        es022_ref[...] = (0.2 * _LOG2E) * es
        edcol_ref[...] = jnp.dot(h, adst_ref[...],
                                 preferred_element_type=jnp.float32)
        ei = ei_ref[...]
        eirow_ref[...] = ei.astype(jnp.int16)
        eicol_ref[...] = jnp.transpose(ei.astype(jnp.float32)).astype(jnp.int16)

    ed = edcol_ref[pl.ds(i * blk, blk), :]
    es2 = es2_ref[...]
    smax2 = jnp.max(es2)
    q2 = _LOG2E * ed + smax2
    mi2 = jnp.maximum(q2, 0.2 * q2)
    edm2 = _LOG2E * ed - mi2
    c22 = (0.2 * _LOG2E) * ed - mi2
    si = eicol_ref[pl.ds(i * blk, blk), 0:1]
    di = eicol_ref[pl.ds(i * blk, blk), 1:2]
    sj = eirow_ref[0:1, :]
    dj = eirow_ref[1:2, :]
    zs = jnp.maximum(edm2 + es2, c22 + es022_ref[...])
    conn = ((si == sj) | (si == dj)) | ((di == sj) | (di == dj))
    p = jnp.where(conn, jnp.exp2(zs).astype(jnp.bfloat16), jnp.bfloat16(0.0))
    acc = jnp.dot(p, hbf_ref[...],
                  preferred_element_type=jnp.float32)
    s = acc[:, C:C + 1]
    out_ref[...] = acc[:, :C] * (1.0 / s) + b_ref[...]


def kernel(x, edge_index, W, a_src, a_dst, b):
    E, _ = x.shape
    C = W.shape[1]
    blk = 1024
    out = pl.pallas_call(
        _gat_kernel,
        grid=(E // blk,),
        in_specs=[
            pl.BlockSpec((E, x.shape[1]), lambda i: (0, 0)),
            pl.BlockSpec((x.shape[1], C), lambda i: (0, 0)),
            pl.BlockSpec((1, C), lambda i: (0, 0)),
            pl.BlockSpec((C, 1), lambda i: (0, 0)),
            pl.BlockSpec((2, E), lambda i: (0, 0)),
            pl.BlockSpec((1, C), lambda i: (0, 0)),
        ],
        out_specs=pl.BlockSpec((blk, C), lambda i: (i, 0)),
        out_shape=jax.ShapeDtypeStruct((E, C), jnp.float32),
        scratch_shapes=[
            pltpu.VMEM((E, C + 128), jnp.bfloat16),
            pltpu.VMEM((1, E), jnp.float32),
            pltpu.VMEM((1, E), jnp.float32),
            pltpu.VMEM((E, 1), jnp.float32),
            pltpu.VMEM((2, E), jnp.int16),
            pltpu.VMEM((E, 2), jnp.int16),
        ],
    )(x, W, a_src.reshape(1, C), a_dst.reshape(C, 1), edge_index,
      b.reshape(1, C))
    return out
